# baseline (speedup 1.0000x reference)
import functools

import numpy as np
import jax
import jax.numpy as jnp
from jax import lax
from jax.experimental import pallas as pl
from jax.experimental.pallas import tpu as pltpu
from jax.experimental.pallas import tpu_sc as plsc

_NC = 2
_NS = 16
_NW = _NC * _NS
_LANES = 16

_CHUNK = 80
_NSLOT = 3


def _positional_encoding_np(seq_len: int, d_model: int) -> np.ndarray:
    position = np.arange(seq_len, dtype=np.float32)[:, None]
    div_term = np.exp(
        np.arange(0, d_model, 2, dtype=np.float32) * (-(np.log(10000.0) / d_model))
    )
    pe = np.zeros((seq_len, d_model), dtype=np.float32)
    pe[:, 0::2] = np.sin(position * div_term)
    pe[:, 1::2] = np.cos(position * div_term)
    return pe


def _sc_embed(tok_idx, comb_idx, token_table, comb_table):
    n_chunks, per_w = tok_idx.shape[1], tok_idx.shape[1] * _CHUNK
    d = token_table.shape[1]
    n = _NW * per_w
    n_comb = comb_table.shape[0]
    n_ring = (n_chunks // _NSLOT) * _NSLOT

    mesh = plsc.VectorSubcoreMesh(core_axis_name="c", subcore_axis_name="s")

    @functools.partial(
        pl.kernel,
        mesh=mesh,
        out_type=jax.ShapeDtypeStruct((n, d), jnp.float32),
        scratch_types=[
            pltpu.VMEM((n_chunks, _CHUNK), jnp.int32),
            pltpu.VMEM((n_chunks, _CHUNK), jnp.int32),
            pltpu.VMEM((n_comb, d), jnp.float32),
        ]
        + [pltpu.VMEM((_CHUNK, d), jnp.float32)] * _NSLOT
        + [pltpu.SemaphoreType.DMA] * (2 * _NSLOT),
    )
    def k(tok_idx_hbm, comb_idx_hbm, table_hbm, comb_hbm, out_hbm,
          tidx_v, cidx_v, comb_v, *bufs_and_sems):
        toks = bufs_and_sems[0:_NSLOT]
        gsems = bufs_and_sems[_NSLOT:2 * _NSLOT]
        osems = bufs_and_sems[2 * _NSLOT:]
        wid = lax.axis_index("s") * _NC + lax.axis_index("c")
        base = wid * per_w
        pltpu.sync_copy(tok_idx_hbm.at[wid], tidx_v)
        pltpu.sync_copy(comb_idx_hbm.at[wid], cidx_v)

        def issue_g(c, s):
            pltpu.async_copy(table_hbm.at[tidx_v.at[c]], toks[s], gsems[s])

        def wait_g(s):
            pltpu.make_async_copy(
                table_hbm.at[tidx_v.at[0]], toks[s], gsems[s]).wait()

        def start_o(c, s):
            pltpu.async_copy(
                toks[s], out_hbm.at[pl.ds(base + c * _CHUNK, _CHUNK)], osems[s])

        def wait_o(s):
            pltpu.make_async_copy(
                toks[s], out_hbm.at[pl.ds(0, _CHUNK)], osems[s]).wait()

        issue_g(0, 0)
        issue_g(1, 1)
        pltpu.sync_copy(comb_hbm, comb_v)

        def add_chunk(c, tok_b):
            def add_body(g, carry):
                cvec = cidx_v[c, pl.ds(g * _LANES, _LANES)]
                cvals = [cvec[r] for r in range(_LANES)]
                for r in range(_LANES):
                    i = g * _LANES + r
                    crow = comb_v.at[cvals[r]]
                    trow = tok_b.at[i]
                    for kk in range(d // _LANES):
                        sl = pl.ds(kk * _LANES, _LANES)
                        plsc.addupdate(trow.at[sl], crow[sl])
                return carry
            lax.fori_loop(0, _CHUNK // _LANES, add_body, 0)

        def process(c, s):
            wait_g(s)
            add_chunk(c, toks[s])
            start_o(c, s)

            @pl.when((c >= 1) & (c + 2 < n_chunks))
            def _():
                wait_o((s + 2) % _NSLOT)

            @pl.when(c + 2 < n_chunks)
            def _():
                issue_g(c + 2, (s + 2) % _NSLOT)

        def ring_body(q, carry):
            for b in range(_NSLOT):
                process(_NSLOT * q + b, b)
            return carry

        lax.fori_loop(0, n_chunks // _NSLOT, ring_body, 0)
        for c in range(n_ring, n_chunks):
            s = c % _NSLOT
            wait_g(s)
            add_chunk(c, toks[s])
            start_o(c, s)
        for s in range(_NSLOT):
            wait_o(s)

    return k(tok_idx, comb_idx, token_table, comb_table)


def kernel(sequence, segment_label, token_table, segment_table):
    b, l = sequence.shape
    d = token_table.shape[1]
    n = b * l

    pe = jnp.asarray(_positional_encoding_np(l, d))
    comb = (pe[:, None, :] + segment_table[None, :, :]).reshape(l * 3, d)

    pos3 = (jnp.arange(l, dtype=jnp.int32) * 3)[None, :]
    comb_idx = (pos3 + segment_label.astype(jnp.int32)).reshape(n)
    tok_idx = sequence.astype(jnp.int32).reshape(n)

    rows_per_w = n // _NW
    tok_idx = tok_idx.reshape(_NW, rows_per_w // _CHUNK, _CHUNK)
    comb_idx = comb_idx.reshape(_NW, rows_per_w // _CHUNK, _CHUNK)

    out = _sc_embed(tok_idx, comb_idx, token_table, comb)
    return out.reshape(b, l, d)


# speedup vs baseline: 1.0251x; 1.0251x over previous
import functools

import numpy as np
import jax
import jax.numpy as jnp
from jax import lax
from jax.experimental import pallas as pl
from jax.experimental.pallas import tpu as pltpu
from jax.experimental.pallas import tpu_sc as plsc

_NC = 2
_NS = 16
_NW = _NC * _NS
_LANES = 16

_CHUNK = 80
_NSLOT = 3


def _positional_encoding_np(seq_len: int, d_model: int) -> np.ndarray:
    position = np.arange(seq_len, dtype=np.float32)[:, None]
    div_term = np.exp(
        np.arange(0, d_model, 2, dtype=np.float32) * (-(np.log(10000.0) / d_model))
    )
    pe = np.zeros((seq_len, d_model), dtype=np.float32)
    pe[:, 0::2] = np.sin(position * div_term)
    pe[:, 1::2] = np.cos(position * div_term)
    return pe


def _sc_embed(tok_idx, comb_idx, token_table, comb_table):
    n_chunks, per_w = tok_idx.shape[1], tok_idx.shape[1] * _CHUNK
    d = token_table.shape[1]
    n = _NW * per_w
    n_comb = comb_table.shape[0]
    n_ring = (n_chunks // _NSLOT) * _NSLOT

    mesh = plsc.VectorSubcoreMesh(core_axis_name="c", subcore_axis_name="s")

    @functools.partial(
        pl.kernel,
        mesh=mesh,
        out_type=jax.ShapeDtypeStruct((n, d), jnp.float32),
        scratch_types=[
            pltpu.VMEM((n_chunks, _CHUNK), jnp.int32),
            pltpu.VMEM((n_chunks, _CHUNK), jnp.int32),
            pltpu.VMEM((n_comb, d), jnp.float32),
        ]
        + [pltpu.VMEM((_CHUNK, d), jnp.float32)] * _NSLOT
        + [pltpu.SemaphoreType.DMA] * (2 * _NSLOT),
    )
    def k(tok_idx_hbm, comb_idx_hbm, table_hbm, comb_hbm, out_hbm,
          tidx_v, cidx_v, comb_v, *bufs_and_sems):
        toks = bufs_and_sems[0:_NSLOT]
        gsems = bufs_and_sems[_NSLOT:2 * _NSLOT]
        osems = bufs_and_sems[2 * _NSLOT:]
        wid = lax.axis_index("s") * _NC + lax.axis_index("c")
        base = wid * per_w
        pltpu.sync_copy(tok_idx_hbm.at[wid], tidx_v)
        pltpu.sync_copy(comb_idx_hbm.at[wid], cidx_v)

        def issue_g(c, s):
            pltpu.async_copy(table_hbm.at[tidx_v.at[c]], toks[s], gsems[s])

        def wait_g(s):
            pltpu.make_async_copy(
                table_hbm.at[tidx_v.at[0]], toks[s], gsems[s]).wait()

        def start_o(c, s):
            pltpu.async_copy(
                toks[s], out_hbm.at[pl.ds(base + c * _CHUNK, _CHUNK)], osems[s])

        def wait_o(s):
            pltpu.make_async_copy(
                toks[s], out_hbm.at[pl.ds(0, _CHUNK)], osems[s]).wait()

        issue_g(0, 0)
        issue_g(1, 1)
        pltpu.sync_copy(comb_hbm, comb_v)

        def add_chunk(c, tok_b):
            @plsc.parallel_loop(0, _CHUNK // _LANES, unroll=2)
            def add_body(g):
                cvec = cidx_v[c, pl.ds(g * _LANES, _LANES)]
                cvals = [cvec[r] for r in range(_LANES)]
                for r in range(_LANES):
                    i = g * _LANES + r
                    crow = comb_v.at[cvals[r]]
                    trow = tok_b.at[i]
                    for kk in range(d // _LANES):
                        sl = pl.ds(kk * _LANES, _LANES)
                        plsc.addupdate(trow.at[sl], crow[sl])

        def process(c, s):
            wait_g(s)
            add_chunk(c, toks[s])
            start_o(c, s)

            @pl.when((c >= 1) & (c + 2 < n_chunks))
            def _():
                wait_o((s + 2) % _NSLOT)

            @pl.when(c + 2 < n_chunks)
            def _():
                issue_g(c + 2, (s + 2) % _NSLOT)

        def ring_body(q, carry):
            for b in range(_NSLOT):
                process(_NSLOT * q + b, b)
            return carry

        lax.fori_loop(0, n_chunks // _NSLOT, ring_body, 0)
        for c in range(n_ring, n_chunks):
            s = c % _NSLOT
            wait_g(s)
            add_chunk(c, toks[s])
            start_o(c, s)
        for s in range(_NSLOT):
            wait_o(s)

    return k(tok_idx, comb_idx, token_table, comb_table)


def kernel(sequence, segment_label, token_table, segment_table):
    b, l = sequence.shape
    d = token_table.shape[1]
    n = b * l

    pe = jnp.asarray(_positional_encoding_np(l, d))
    comb = (pe[:, None, :] + segment_table[None, :, :]).reshape(l * 3, d)

    pos3 = (jnp.arange(l, dtype=jnp.int32) * 3)[None, :]
    comb_idx = (pos3 + segment_label.astype(jnp.int32)).reshape(n)
    tok_idx = sequence.astype(jnp.int32).reshape(n)

    rows_per_w = n // _NW
    tok_idx = tok_idx.reshape(_NW, rows_per_w // _CHUNK, _CHUNK)
    comb_idx = comb_idx.reshape(_NW, rows_per_w // _CHUNK, _CHUNK)

    out = _sc_embed(tok_idx, comb_idx, token_table, comb)
    return out.reshape(b, l, d)
